# baseline (device time: 409033 ns/iter reference)
import jax
import jax.numpy as jnp
from jax import lax
from jax.experimental import pallas as pl
from jax.experimental.pallas import tpu as pltpu

_TILES = 8


def kernel(x):
    m, n = x.shape
    half = n // 2
    rows = m // _TILES

    def body(x_ref, out_ref, vbuf, in_sems, out_sems, send_sem, recv_sem):
        my_x = lax.axis_index("x")
        my_y = lax.axis_index("y")
        my_z = lax.axis_index("z")
        peer_y = 1 - my_y

        barrier = pltpu.get_barrier_semaphore()
        pl.semaphore_signal(
            barrier,
            inc=1,
            device_id=(my_x, peer_y, my_z),
            device_id_type=pl.DeviceIdType.MESH,
        )
        pl.semaphore_wait(barrier, 1)

        rdma = pltpu.make_async_remote_copy(
            src_ref=x_ref.at[:, pl.ds(peer_y * half, half)],
            dst_ref=out_ref.at[pl.ds(my_y * m, m), :],
            send_sem=send_sem,
            recv_sem=recv_sem,
            device_id=(my_x, peer_y, my_z),
            device_id_type=pl.DeviceIdType.MESH,
        )
        rdma.start()

        stores = []
        for i in range(_TILES):
            s = i % 2
            if i >= 2:
                stores[i - 2].wait()
            load = pltpu.make_async_copy(
                x_ref.at[pl.ds(i * rows, rows), pl.ds(my_y * half, half)],
                vbuf.at[s],
                in_sems.at[s],
            )
            load.start()
            load.wait()
            store = pltpu.make_async_copy(
                vbuf.at[s],
                out_ref.at[pl.ds(my_y * m + i * rows, rows), :],
                out_sems.at[s],
            )
            store.start()
            stores.append(store)
        stores[-2].wait()
        stores[-1].wait()

        rdma.wait()

    return pl.pallas_call(
        body,
        out_shape=jax.ShapeDtypeStruct((2 * m, half), jnp.float32),
        in_specs=[pl.BlockSpec(memory_space=pl.ANY)],
        out_specs=pl.BlockSpec(memory_space=pl.ANY),
        scratch_shapes=[
            pltpu.VMEM((2, rows, half), jnp.float32),
            pltpu.SemaphoreType.DMA((2,)),
            pltpu.SemaphoreType.DMA((2,)),
            pltpu.SemaphoreType.DMA,
            pltpu.SemaphoreType.DMA,
        ],
        compiler_params=pltpu.CompilerParams(collective_id=0),
    )(x)
